# baseline (speedup 1.0000x reference)
import functools

import jax
import jax.numpy as jnp
from jax import lax
from jax.experimental import pallas as pl
from jax.experimental.pallas import tpu as pltpu
from jax.experimental.pallas import tpu_sc as plsc

_B = 16384
_H = 50
_D = 32
_V = 1000000

_NW = 32
_TB = _V // 256
_TAIL = _V - _TB * 256
_NQUAD = (_TB + 4 * _NW - 1) // (4 * _NW)

_BPW = _B // _NW
_CPW = _BPW // 4


def _make_transpose():
  mesh = plsc.VectorSubcoreMesh(core_axis_name="c", subcore_axis_name="s")

  @functools.partial(
      pl.kernel,
      mesh=mesh,
      out_type=jax.ShapeDtypeStruct((_V * _D,), jnp.float32),
      scratch_types=(
          [pltpu.VMEM((32, 256), jnp.float32) for _ in range(4)]
          + [pltpu.VMEM((8192,), jnp.float32) for _ in range(4)]
          + [pltpu.VMEM((32, 64), jnp.float32), pltpu.VMEM((2048,), jnp.float32)]
          + [pltpu.SemaphoreType.DMA for _ in range(8)]
      ),
      compiler_params=pltpu.CompilerParams(use_tc_tiling_on_sc=True,
                                           needs_layout_passes=False,
                                           disable_bounds_checks=True),
  )
  def tk(tt_hbm, t2_hbm, g0, g1, g2, g3, b0, b1, b2, b3, gtail, stail,
         si0, si1, si2, si3, so0, so1, so2, so3):
    gbuf = (g0, g1, g2, g3)
    sbuf = (b0, b1, b2, b3)
    si = (si0, si1, si2, si3)
    so = (so0, so1, so2, so3)
    wid = lax.axis_index("s") * 2 + lax.axis_index("c")
    iota = lax.iota(jnp.int32, 16)
    pat = [((16 * m + iota) // 4) * 128 + ((16 * m + iota) % 4) * 32
           for m in range(16)]

    def in_start(j, s):
      pltpu.async_copy(tt_hbm.at[:, pl.ds(j * 256, 256)], gbuf[s], si[s])

    def in_wait(j, s):
      pltpu.make_async_copy(tt_hbm.at[:, pl.ds(j * 256, 256)], gbuf[s],
                            si[s]).wait()

    def out_start(j, s):
      pltpu.async_copy(sbuf[s], t2_hbm.at[pl.ds(j * 8192, 8192)], so[s])

    def out_wait(s):
      pltpu.make_async_copy(sbuf[s], t2_hbm.at[pl.ds(0, 8192)], so[s]).wait()

    def vec(s):
      g, sb = gbuf[s], sbuf[s]
      for e in range(32):
        vals = [g[e, pl.ds(16 * m, 16)] for m in range(16)]
        for m in range(16):
          plsc.store_scatter(sb, [pat[m] + e], vals[m])

    for s in range(4):
      in_start(wid + _NW * s, s)

    def blk4(ii, carry):
      for s in range(4):
        jj = 4 * ii + s
        j = wid + _NW * jj

        @pl.when(j < _TB)
        def _():
          in_wait(j, s)

          @pl.when(jj >= 4)
          def _():
            out_wait(s)

          vec(s)
          out_start(j, s)

          @pl.when(j + 4 * _NW < _TB)
          def _():
            in_start(j + 4 * _NW, s)

      return carry

    lax.fori_loop(0, _NQUAD, blk4, 0)
    for s in range(4):
      out_wait(s)

    @pl.when(wid == 0)
    def _():
      pltpu.sync_copy(tt_hbm.at[:, pl.ds(_TB * 256, _TAIL)], gtail)
      for e in range(32):
        vals = [gtail[e, pl.ds(16 * m, 16)] for m in range(4)]
        for m in range(4):
          plsc.store_scatter(stail, [pat[m] + e], vals[m])
      pltpu.sync_copy(stail, t2_hbm.at[pl.ds(_TB * 8192, 2048)])

  return tk


def _make_gather():
  mesh = plsc.VectorSubcoreMesh(core_axis_name="c", subcore_axis_name="s")

  @functools.partial(
      pl.kernel,
      mesh=mesh,
      out_type=jax.ShapeDtypeStruct((_H, 4, 131072), jnp.float32),
      scratch_types=(
          [pltpu.VMEM((_CPW,), jnp.int32) for _ in range(4)]
          + [pltpu.VMEM((_CPW, _D), jnp.float32) for _ in range(4)]
          + [pltpu.VMEM((4096,), jnp.float32) for _ in range(4)]
          + [pltpu.SemaphoreType.DMA for _ in range(12)]
      ),
      compiler_params=pltpu.CompilerParams(use_tc_tiling_on_sc=False,
                                           needs_layout_passes=False,
                                           disable_bounds_checks=True),
  )
  def gk(t_hbm, idx_hbm, out_hbm, i0, i1, i2, i3, r0, r1, r2, r3,
         t0_, t1_, t2_, t3_, su0, su1, su2, su3, sg0, sg1, sg2, sg3,
         sn0, sn1, sn2, sn3):
    idxv = (i0, i1, i2, i3)
    rows = (r0, r1, r2, r3)
    stg = (t0_, t1_, t2_, t3_)
    su = (su0, su1, su2, su3)
    sg = (sg0, sg1, sg2, sg3)
    so = (sn0, sn1, sn2, sn3)
    wid = lax.axis_index("s") * 2 + lax.axis_index("c")
    iota = lax.iota(jnp.int32, 16)
    qpat = [((16 * q + iota) // 8) * 1024 + ((16 * q + iota) % 8) * 128
            for q in range(2)]
    njobs = 4 * _H

    def idx_start(jo, s):
      off = (jo // 4) * _B + wid * _BPW + (jo % 4) * _CPW
      pltpu.async_copy(idx_hbm.at[pl.ds(off, _CPW)], idxv[s], su[s])

    def idx_wait(s):
      pltpu.make_async_copy(idx_hbm.at[pl.ds(0, _CPW)], idxv[s], su[s]).wait()

    def g_start(s):
      pltpu.async_copy(t_hbm.at[idxv[s]], rows[s], sg[s])

    def g_wait(s):
      pltpu.make_async_copy(t_hbm.at[idxv[s]], rows[s], sg[s]).wait()

    def out_start(jo, s):
      h = jo // 4
      qoff = 4096 * wid + (jo % 4) * 1024
      for i in range(4):
        pltpu.async_copy(stg[s].at[pl.ds(i * 1024, 1024)],
                         out_hbm.at[h, i, pl.ds(qoff, 1024)], so[s])

    def out_wait(s):
      for i in range(4):
        pltpu.make_async_copy(stg[s].at[pl.ds(i * 1024, 1024)],
                              out_hbm.at[0, i, pl.ds(0, 1024)], so[s]).wait()

    def vec(s):
      r, sb = rows[s], stg[s]
      for t0 in range(0, _CPW, 4):
        vals = []
        for dt in range(4):
          for q in range(2):
            vals.append(r[t0 + dt, pl.ds(16 * q, 16)])
        vi = 0
        for dt in range(4):
          toff = t0 + dt
          for q in range(2):
            plsc.store_scatter(sb, [qpat[q] + toff], vals[vi])
            vi += 1

    for s in range(4):
      idx_start(s, s)
    idx_wait(0)
    g_start(0)
    idx_wait(1)
    g_start(1)

    def hloop(ii, carry):
      for s in range(4):
        jo = 4 * ii + s
        s2 = (s + 2) % 4
        g_wait(s)

        @pl.when(jo + 4 < njobs)
        def _():
          idx_start(jo + 4, s)

        @pl.when(jo + 2 < njobs)
        def _():
          idx_wait(s2)
          g_start(s2)

        @pl.when(jo >= 4)
        def _():
          out_wait(s)

        vec(s)
        out_start(jo, s)
      return carry

    lax.fori_loop(0, njobs // 4, hloop, 0)
    for s in range(4):
      out_wait(s)

  return gk


_transpose = _make_transpose()
_gather = _make_gather()


def kernel(x, table):
  t2 = _transpose(table.T)
  t_lin = t2.reshape(_V, _D)
  idxT = x.T.reshape(_B * _H)
  out7 = _gather(t_lin, idxT)
  out5 = out7.reshape(_H, 4, 128, 8, 128)
  return out5.transpose(2, 4, 0, 1, 3).reshape(_B, _H, _D)


# speedup vs baseline: 1.2751x; 1.2751x over previous
import functools

import jax
import jax.numpy as jnp
from jax import lax
from jax.experimental import pallas as pl
from jax.experimental.pallas import tpu as pltpu
from jax.experimental.pallas import tpu_sc as plsc

_B = 16384
_H = 50
_D = 32
_V = 1000000

_NW = 32
_TB = _V // 256
_TAIL = _V - _TB * 256
_NQUAD = (_TB + 4 * _NW - 1) // (4 * _NW)

_BPW = _B // _NW
_CPW = _BPW // 4


def _make_transpose():
  mesh = plsc.VectorSubcoreMesh(core_axis_name="c", subcore_axis_name="s")

  @functools.partial(
      pl.kernel,
      mesh=mesh,
      out_type=jax.ShapeDtypeStruct((_V * _D,), jnp.float32),
      scratch_types=(
          [pltpu.VMEM((32, 256), jnp.float32) for _ in range(4)]
          + [pltpu.VMEM((8192,), jnp.float32) for _ in range(4)]
          + [pltpu.VMEM((32, 64), jnp.float32), pltpu.VMEM((2048,), jnp.float32)]
          + [pltpu.SemaphoreType.DMA for _ in range(8)]
      ),
      compiler_params=pltpu.CompilerParams(use_tc_tiling_on_sc=True,
                                           needs_layout_passes=False,
                                           disable_bounds_checks=True),
  )
  def tk(tt_hbm, t2_hbm, g0, g1, g2, g3, b0, b1, b2, b3, gtail, stail,
         si0, si1, si2, si3, so0, so1, so2, so3):
    gbuf = (g0, g1, g2, g3)
    sbuf = (b0, b1, b2, b3)
    si = (si0, si1, si2, si3)
    so = (so0, so1, so2, so3)
    wid = lax.axis_index("s") * 2 + lax.axis_index("c")
    iota = lax.iota(jnp.int32, 16)
    pat = [((16 * m + iota) // 4) * 128 + ((16 * m + iota) % 4) * 32
           for m in range(16)]

    def in_start(j, s):
      pltpu.async_copy(tt_hbm.at[:, pl.ds(j * 256, 256)], gbuf[s], si[s])

    def in_wait(j, s):
      pltpu.make_async_copy(tt_hbm.at[:, pl.ds(j * 256, 256)], gbuf[s],
                            si[s]).wait()

    def out_start(j, s):
      pltpu.async_copy(sbuf[s], t2_hbm.at[pl.ds(j * 8192, 8192)], so[s])

    def out_wait(s):
      pltpu.make_async_copy(sbuf[s], t2_hbm.at[pl.ds(0, 8192)], so[s]).wait()

    def vec(s):
      g, sb = gbuf[s], sbuf[s]
      for e in range(32):
        vals = [g[e, pl.ds(16 * m, 16)] for m in range(16)]
        for m in range(16):
          plsc.store_scatter(sb, [pat[m] + e], vals[m])

    for s in range(4):
      in_start(wid + _NW * s, s)

    def blk4(ii, carry):
      for s in range(4):
        jj = 4 * ii + s
        j = wid + _NW * jj

        @pl.when(j < _TB)
        def _():
          in_wait(j, s)

          @pl.when(jj >= 4)
          def _():
            out_wait(s)

          vec(s)
          out_start(j, s)

          @pl.when(j + 4 * _NW < _TB)
          def _():
            in_start(j + 4 * _NW, s)

      return carry

    lax.fori_loop(0, _NQUAD, blk4, 0)
    for s in range(4):
      out_wait(s)

    @pl.when(wid == 0)
    def _():
      pltpu.sync_copy(tt_hbm.at[:, pl.ds(_TB * 256, _TAIL)], gtail)
      for e in range(32):
        vals = [gtail[e, pl.ds(16 * m, 16)] for m in range(4)]
        for m in range(4):
          plsc.store_scatter(stail, [pat[m] + e], vals[m])
      pltpu.sync_copy(stail, t2_hbm.at[pl.ds(_TB * 8192, 2048)])

  return tk


def _make_gather():
  mesh = plsc.VectorSubcoreMesh(core_axis_name="c", subcore_axis_name="s")

  @functools.partial(
      pl.kernel,
      mesh=mesh,
      out_type=jax.ShapeDtypeStruct((_H, 4, 128, 8, 128), jnp.float32),
      scratch_types=(
          [pltpu.VMEM((_CPW,), jnp.int32) for _ in range(4)]
          + [pltpu.VMEM((_CPW, _D), jnp.float32) for _ in range(4)]
          + [pltpu.VMEM((32, 135), jnp.float32) for _ in range(4)]
          + [pltpu.SemaphoreType.DMA for _ in range(12)]
      ),
      compiler_params=pltpu.CompilerParams(use_tc_tiling_on_sc=False,
                                           needs_layout_passes=False,
                                           disable_bounds_checks=True),
  )
  def gk(t_hbm, idx_hbm, out_hbm, i0, i1, i2, i3, r0, r1, r2, r3,
         t0_, t1_, t2_, t3_, su0, su1, su2, su3, sg0, sg1, sg2, sg3,
         sn0, sn1, sn2, sn3):
    idxv = (i0, i1, i2, i3)
    rows = (r0, r1, r2, r3)
    stg = (t0_, t1_, t2_, t3_)
    su = (su0, su1, su2, su3)
    sg = (sg0, sg1, sg2, sg3)
    so = (sn0, sn1, sn2, sn3)
    wid = lax.axis_index("s") * 2 + lax.axis_index("c")
    iota = lax.iota(jnp.int32, 16)
    qpat = [16 * q + iota for q in range(2)]
    njobs = 4 * _H

    def idx_start(jo, s):
      off = (jo // 4) * _B + wid * _BPW + (jo % 4) * _CPW
      pltpu.async_copy(idx_hbm.at[pl.ds(off, _CPW)], idxv[s], su[s])

    def idx_wait(s):
      pltpu.make_async_copy(idx_hbm.at[pl.ds(0, _CPW)], idxv[s], su[s]).wait()

    def g_start(s):
      pltpu.async_copy(t_hbm.at[idxv[s]], rows[s], sg[s])

    def g_wait(s):
      pltpu.make_async_copy(t_hbm.at[idxv[s]], rows[s], sg[s]).wait()

    def out_start(jo, s):
      h = jo // 4
      jg = 4 * wid + jo % 4
      for i in range(4):
        pltpu.async_copy(stg[s].at[pl.ds(8 * i, 8), pl.ds(0, 128)],
                         out_hbm.at[h, i, jg], so[s])

    def out_wait(s):
      for i in range(4):
        pltpu.make_async_copy(stg[s].at[pl.ds(8 * i, 8), pl.ds(0, 128)],
                              out_hbm.at[0, i, 0], so[s]).wait()

    def vec(s):
      r, sb = rows[s], stg[s]
      for t0 in range(0, _CPW, 4):
        vals = []
        for dt in range(4):
          for q in range(2):
            vals.append(r[t0 + dt, pl.ds(16 * q, 16)])
        vi = 0
        for dt in range(4):
          tv = jnp.full((16,), t0 + dt, jnp.int32)
          for q in range(2):
            plsc.store_scatter(sb, [qpat[q], tv], vals[vi])
            vi += 1

    for s in range(4):
      idx_start(s, s)
    idx_wait(0)
    g_start(0)
    idx_wait(1)
    g_start(1)

    def hloop(ii, carry):
      for s in range(4):
        jo = 4 * ii + s
        s2 = (s + 2) % 4
        g_wait(s)

        @pl.when(jo + 4 < njobs)
        def _():
          idx_start(jo + 4, s)

        @pl.when(jo + 2 < njobs)
        def _():
          idx_wait(s2)
          g_start(s2)

        @pl.when(jo >= 4)
        def _():
          out_wait(s)

        vec(s)
        out_start(jo, s)
      return carry

    lax.fori_loop(0, njobs // 4, hloop, 0)
    for s in range(4):
      out_wait(s)

  return gk


_transpose = _make_transpose()
_gather = _make_gather()


def kernel(x, table):
  t2 = _transpose(table.T)
  t_lin = t2.reshape(_V, _D)
  idxT = x.T.reshape(_B * _H)
  out5 = _gather(t_lin, idxT)
  return out5.transpose(2, 4, 0, 1, 3).reshape(_B, _H, _D)
